# baseline (speedup 1.0000x reference)
import jax
import jax.numpy as jnp
from jax import lax
from jax.experimental import pallas as pl
from jax.experimental.pallas import tpu as pltpu
from jax.experimental.pallas import tpu_sc as plsc

N_NODES = 10000
NP = 10240
DIM = 128
HD = DIM // 2
NC, NS = 2, 16
NW = NC * NS
CH = 128
ROWS_PER_TILE = NP // NS
NBUF = 8


def _sc_mesh():
    return plsc.VectorSubcoreMesh(
        core_axis_name="c", subcore_axis_name="s",
        num_cores=NC, num_subcores=NS)


def _make_agg(e_pad, d, split_features, with_count):
    if split_features:
        chunks_per_worker = e_pad // (NS * CH)
    else:
        chunks_per_worker = e_pad // (NW * CH)
    nsuper = chunks_per_worker // NBUF

    def body(*refs):
        if with_count:
            (src_hbm, dst_hbm, x_hbm, zrows_hbm, zcnt_hbm, ones_hbm,
             out_hbm, cnt_hbm,
             sidx, didx, rows, ones, acc_s, cnt_s, gsem, ssem, csem) = refs
        else:
            (src_hbm, dst_hbm, x_hbm, zrows_hbm,
             out_hbm,
             sidx, didx, rows, acc_s, gsem, ssem) = refs

        cid = lax.axis_index("c")
        sid = lax.axis_index("s")
        wid = sid * NC + cid
        r_tile = sid * ROWS_PER_TILE

        pltpu.sync_copy(zrows_hbm, acc_s.at[pl.ds(r_tile, ROWS_PER_TILE)])
        if with_count:
            pltpu.sync_copy(zcnt_hbm, cnt_s.at[pl.ds(r_tile, ROWS_PER_TILE)])
            pltpu.sync_copy(ones_hbm, ones)

        base0 = (sid if split_features else wid) * chunks_per_worker

        def load_idx(s, parity):
            row0 = base0 + s * NBUF
            plane = cid if split_features else 0
            pltpu.sync_copy(src_hbm.at[plane, pl.ds(row0, NBUF)],
                            sidx.at[parity])
            pltpu.sync_copy(dst_hbm.at[pl.ds(row0, NBUF)], didx.at[parity])

        def gather(parity, b):
            return pltpu.make_async_copy(
                x_hbm.at[sidx.at[parity, b]], rows.at[b], gsem.at[b])

        def scatter(parity, b):
            return pltpu.make_async_copy(
                rows.at[b], acc_s.at[didx.at[parity, b]], ssem.at[b])

        def cscatter(parity, b):
            return pltpu.make_async_copy(
                ones, cnt_s.at[didx.at[parity, b]], csem.at[b])

        load_idx(0, 0)
        for b in range(NBUF):
            gather(0, b).start()
        plsc.subcore_barrier()

        def superstep(s, _):
            p = lax.rem(s, 2)
            np_ = 1 - p

            @pl.when(s < nsuper - 1)
            def _():
                load_idx(s + 1, np_)

            for b in range(NBUF):
                gather(p, b).wait()
                scatter(p, b).start(add=True)
                if with_count:
                    @pl.when(cid == 0)
                    def _():
                        cscatter(p, b).start(add=True)
            for b in range(NBUF):
                scatter(p, b).wait()
                if with_count:
                    @pl.when(cid == 0)
                    def _():
                        cscatter(p, b).wait()

                @pl.when(s < nsuper - 1)
                def _():
                    gather(np_, b).start()
            return 0
        lax.fori_loop(0, nsuper, superstep, 0)
        plsc.subcore_barrier()

        pltpu.sync_copy(acc_s.at[pl.ds(r_tile, ROWS_PER_TILE)],
                        out_hbm.at[pl.ds(r_tile, ROWS_PER_TILE), cid])
        if with_count:
            @pl.when(cid == 0)
            def _():
                pltpu.sync_copy(cnt_s.at[pl.ds(r_tile, ROWS_PER_TILE)],
                                cnt_hbm.at[pl.ds(r_tile, ROWS_PER_TILE)])

    out_type = [jax.ShapeDtypeStruct((NP, NC, d), jnp.float32)]
    scratch = [
        pltpu.VMEM((2, NBUF, CH), jnp.int32),
        pltpu.VMEM((2, NBUF, CH), jnp.int32),
        pltpu.VMEM((NBUF, CH, d), jnp.float32),
    ]
    if with_count:
        out_type.append(jax.ShapeDtypeStruct((NP, 16), jnp.float32))
        scratch.append(pltpu.VMEM((CH, 16), jnp.float32))
    scratch.append(pltpu.VMEM_SHARED((NP, d), jnp.float32))
    if with_count:
        scratch.append(pltpu.VMEM_SHARED((NP, 16), jnp.float32))
    scratch.append(pltpu.SemaphoreType.DMA((NBUF,)))
    scratch.append(pltpu.SemaphoreType.DMA((NBUF,)))
    if with_count:
        scratch.append(pltpu.SemaphoreType.DMA((NBUF,)))

    return pl.kernel(body, out_type=out_type, mesh=_sc_mesh(),
                     scratch_types=scratch,
                     compiler_params=pltpu.CompilerParams(
                         use_tc_tiling_on_sc=False))


NB = 1024
GRID = NP // NB


def _invd(cnt_ref):
    deg = cnt_ref[:, 0:1]
    return 1.0 / jnp.maximum(deg, 1.0)


def _tc_layer12(p_ref, cnt_ref, h_ref, wl_ref, wr_ref, b_ref, o_ref):
    agg = p_ref[...] * _invd(cnt_ref)
    h = (jnp.dot(agg, wl_ref[...], preferred_element_type=jnp.float32)
         + jnp.dot(h_ref[...], wr_ref[...], preferred_element_type=jnp.float32)
         + b_ref[...])
    o_ref[...] = jnp.maximum(h, 0.0)


def _tc_layer2b(p_ref, cnt_ref, h_ref, wl_ref, wr_ref, b_ref, w3l_ref, w3r_ref,
                y_ref, z_ref):
    agg = p_ref[...] * _invd(cnt_ref)
    h2 = (jnp.dot(agg, wl_ref[...], preferred_element_type=jnp.float32)
          + jnp.dot(h_ref[...], wr_ref[...], preferred_element_type=jnp.float32)
          + b_ref[...])
    h2 = jnp.maximum(h2, 0.0)
    y_ref[...] = jnp.dot(h2, w3l_ref[...], preferred_element_type=jnp.float32)
    z_ref[...] = jnp.dot(h2, w3r_ref[...], preferred_element_type=jnp.float32)


def _tc_layer3(a_ref, cnt_ref, z_ref, b_ref, o_ref):
    h3 = ((a_ref[:, 0, 0:1] + a_ref[:, 1, 0:1]) * _invd(cnt_ref)
          + z_ref[:, 0:1] + b_ref[...])
    cols = lax.broadcasted_iota(jnp.int32, (NB, 16), 1)
    o_ref[...] = jnp.where(cols == 0, h3, 0.0)


def _tc_layer4(a_ref, cnt_ref, h3_ref, wl_ref, wr_ref, b_ref, o_ref):
    a4n = (a_ref[:, 0, 0:1] + a_ref[:, 1, 0:1]) * _invd(cnt_ref)
    logits = (a4n * wl_ref[...] + h3_ref[:, 0:1] * wr_ref[...] + b_ref[...])
    m = jnp.max(logits, axis=1, keepdims=True)
    sh = logits - m
    lse = jnp.log(jnp.sum(jnp.exp(sh), axis=1, keepdims=True))
    o_ref[...] = sh - lse


def _spec_a16():
    return pl.BlockSpec((NB, 2, 16), lambda i: (i, 0, 0))


def _spec_rows(w):
    return pl.BlockSpec((NB, w), lambda i: (i, 0))


def _spec_full(shape):
    return pl.BlockSpec(shape, lambda i: tuple(0 for _ in shape))


def kernel(x, edge_index, Wl1, Wr1, b1, Wl2, Wr2, b2, Wl3, Wr3, b3,
           Wl4, Wr4, b4):
    x = x.astype(jnp.float32)
    src = edge_index[0].astype(jnp.int32)
    dst = edge_index[1].astype(jnp.int32)
    e = src.shape[0]
    quantum = NW * CH * NBUF
    e_pad = -(-e // quantum) * quantum
    if e_pad != e:
        src = jnp.concatenate([src, jnp.zeros((e_pad - e,), jnp.int32)])
        dst = jnp.concatenate(
            [dst, jnp.full((e_pad - e,), N_NODES, jnp.int32)])
    src = src.reshape(e_pad // CH, CH)
    dst = dst.reshape(e_pad // CH, CH)
    srcs_w = jnp.stack([2 * src, 2 * src + 1])
    srcs_s = src[None]
    xp = jnp.pad(x, ((0, NP - x.shape[0]), (0, 0)))

    zw = jnp.zeros((ROWS_PER_TILE, HD), jnp.float32)
    z16 = jnp.zeros((ROWS_PER_TILE, 16), jnp.float32)
    ones = jnp.ones((CH, 16), jnp.float32)

    agg_wide_cnt = _make_agg(e_pad, HD, True, True)
    agg_wide = _make_agg(e_pad, HD, True, False)
    agg_16 = _make_agg(e_pad, 16, False, False)

    p1, cnt = agg_wide_cnt(srcs_w, dst, xp.reshape(2 * NP, HD),
                           zw, z16, ones)
    h1 = pl.pallas_call(
        _tc_layer12,
        grid=(GRID,),
        in_specs=[_spec_rows(DIM), _spec_rows(16), _spec_rows(DIM),
                  _spec_full((DIM, DIM)), _spec_full((DIM, DIM)),
                  _spec_full((1, DIM))],
        out_specs=_spec_rows(DIM),
        out_shape=jax.ShapeDtypeStruct((NP, DIM), jnp.float32),
    )(p1.reshape(NP, DIM), cnt, xp, Wl1, Wr1, b1.reshape(1, DIM))

    p2 = agg_wide(srcs_w, dst, h1.reshape(2 * NP, HD), zw)[0]
    w3l = jnp.pad(Wl3, ((0, 0), (0, 15)))
    w3r = jnp.pad(Wr3, ((0, 0), (0, 15)))
    y3w, z3w = pl.pallas_call(
        _tc_layer2b,
        grid=(GRID,),
        in_specs=[_spec_rows(DIM), _spec_rows(16), _spec_rows(DIM),
                  _spec_full((DIM, DIM)), _spec_full((DIM, DIM)),
                  _spec_full((1, DIM)), _spec_full((DIM, 16)),
                  _spec_full((DIM, 16))],
        out_specs=[_spec_rows(16), _spec_rows(16)],
        out_shape=[jax.ShapeDtypeStruct((NP, 16), jnp.float32),
                   jax.ShapeDtypeStruct((NP, 16), jnp.float32)],
    )(p2.reshape(NP, DIM), cnt, h1, Wl2, Wr2, b2.reshape(1, DIM), w3l, w3r)

    a3 = agg_16(srcs_s, dst, y3w, z16)[0]
    h3w = pl.pallas_call(
        _tc_layer3,
        grid=(GRID,),
        in_specs=[_spec_a16(), _spec_rows(16), _spec_rows(16),
                  _spec_full((1, 1))],
        out_specs=_spec_rows(16),
        out_shape=jax.ShapeDtypeStruct((NP, 16), jnp.float32),
    )(a3, cnt, z3w, b3.reshape(1, 1))

    a4 = agg_16(srcs_s, dst, h3w, z16)[0]
    out = pl.pallas_call(
        _tc_layer4,
        grid=(GRID,),
        in_specs=[_spec_a16(), _spec_rows(16), _spec_rows(16),
                  _spec_full((1, 16)), _spec_full((1, 16)),
                  _spec_full((1, 16))],
        out_specs=_spec_rows(16),
        out_shape=jax.ShapeDtypeStruct((NP, 16), jnp.float32),
    )(a4, cnt, h3w, Wl4, Wr4, b4.reshape(1, 16))

    n = x.shape[0]
    return (out[:n], h3w[:n, 0])


# speedup vs baseline: 1.3506x; 1.3506x over previous
import jax
import jax.numpy as jnp
from jax import lax
from jax.experimental import pallas as pl
from jax.experimental.pallas import tpu as pltpu
from jax.experimental.pallas import tpu_sc as plsc

N_NODES = 10000
NP = 10240
DIM = 128
HD = DIM // 2
NC, NS = 2, 16
NW = NC * NS
CH = 128
ROWS_PER_TILE = NP // NS
NBUF = 8


def _sc_mesh():
    return plsc.VectorSubcoreMesh(
        core_axis_name="c", subcore_axis_name="s",
        num_cores=NC, num_subcores=NS)


def _make_agg(e_pad, d, split_features, with_count):
    if split_features:
        chunks_per_worker = e_pad // (NS * CH)
    else:
        chunks_per_worker = e_pad // (NW * CH)
    nsuper = chunks_per_worker // NBUF

    def body(*refs):
        if with_count:
            (src_hbm, dst_hbm, x_hbm, zrows_hbm, zcnt_hbm, ones_hbm,
             out_hbm, cnt_hbm,
             sidx, didx, rows, ones, acc_s, cnt_s, gsem, ssem, csem) = refs
        else:
            (src_hbm, dst_hbm, x_hbm, zrows_hbm,
             out_hbm,
             sidx, didx, rows, acc_s, gsem, ssem) = refs

        cid = lax.axis_index("c")
        sid = lax.axis_index("s")
        wid = sid * NC + cid
        r_tile = sid * ROWS_PER_TILE

        pltpu.sync_copy(zrows_hbm, acc_s.at[pl.ds(r_tile, ROWS_PER_TILE)])
        if with_count:
            pltpu.sync_copy(zcnt_hbm, cnt_s.at[pl.ds(r_tile, ROWS_PER_TILE)])
            pltpu.sync_copy(ones_hbm, ones)

        base0 = (sid if split_features else wid) * chunks_per_worker

        def load_idx(s, parity):
            row0 = base0 + s * NBUF
            plane = cid if split_features else 0
            pltpu.sync_copy(src_hbm.at[plane, pl.ds(row0, NBUF)],
                            sidx.at[parity])
            pltpu.sync_copy(dst_hbm.at[pl.ds(row0, NBUF)], didx.at[parity])

        def gather(parity, b):
            return pltpu.make_async_copy(
                x_hbm.at[sidx.at[parity, b]], rows.at[b], gsem.at[b])

        def scatter(parity, b):
            return pltpu.make_async_copy(
                rows.at[b], acc_s.at[didx.at[parity, b]], ssem.at[b])

        def cscatter(parity, b):
            return pltpu.make_async_copy(
                ones, cnt_s.at[didx.at[parity, b]], csem.at[b])

        load_idx(0, 0)
        for b in range(NBUF):
            gather(0, b).start()
        plsc.subcore_barrier()

        def superstep(s, _):
            p = lax.rem(s, 2)
            np_ = 1 - p

            @pl.when(s < nsuper - 1)
            def _():
                load_idx(s + 1, np_)

            for b in range(NBUF):
                gather(p, b).wait()
                scatter(p, b).start(add=True)
                if with_count:
                    @pl.when(cid == 0)
                    def _():
                        cscatter(p, b).start(add=True)
            for b in range(NBUF):
                scatter(p, b).wait()
                if with_count:
                    @pl.when(cid == 0)
                    def _():
                        cscatter(p, b).wait()

                @pl.when(s < nsuper - 1)
                def _():
                    gather(np_, b).start()
            return 0
        lax.fori_loop(0, nsuper, superstep, 0)
        plsc.subcore_barrier()

        pltpu.sync_copy(acc_s.at[pl.ds(r_tile, ROWS_PER_TILE)],
                        out_hbm.at[pl.ds(cid * NP + r_tile, ROWS_PER_TILE)])
        if with_count:
            @pl.when(cid == 0)
            def _():
                pltpu.sync_copy(cnt_s.at[pl.ds(r_tile, ROWS_PER_TILE)],
                                cnt_hbm.at[pl.ds(r_tile, ROWS_PER_TILE)])

    out_type = [jax.ShapeDtypeStruct((NC * NP, d), jnp.float32)]
    scratch = [
        pltpu.VMEM((2, NBUF, CH), jnp.int32),
        pltpu.VMEM((2, NBUF, CH), jnp.int32),
        pltpu.VMEM((NBUF, CH, d), jnp.float32),
    ]
    if with_count:
        out_type.append(jax.ShapeDtypeStruct((NP, 16), jnp.float32))
        scratch.append(pltpu.VMEM((CH, 16), jnp.float32))
    scratch.append(pltpu.VMEM_SHARED((NP, d), jnp.float32))
    if with_count:
        scratch.append(pltpu.VMEM_SHARED((NP, 16), jnp.float32))
    scratch.append(pltpu.SemaphoreType.DMA((NBUF,)))
    scratch.append(pltpu.SemaphoreType.DMA((NBUF,)))
    if with_count:
        scratch.append(pltpu.SemaphoreType.DMA((NBUF,)))

    return pl.kernel(body, out_type=out_type, mesh=_sc_mesh(),
                     scratch_types=scratch,
                     compiler_params=pltpu.CompilerParams(
                         use_tc_tiling_on_sc=False))


NB = 1024
GRID = NP // NB


def _invd(cnt_ref):
    deg = cnt_ref[:, 0:1]
    return 1.0 / jnp.maximum(deg, 1.0)


def _tc_layer12(p_ref, cnt_ref, h_ref, wl_ref, wr_ref, b_ref, o_ref):
    agg = jnp.concatenate([p_ref[0], p_ref[1]], axis=1) * _invd(cnt_ref)
    h = (jnp.dot(agg, wl_ref[...], preferred_element_type=jnp.float32)
         + jnp.dot(h_ref[...], wr_ref[...], preferred_element_type=jnp.float32)
         + b_ref[...])
    o_ref[...] = jnp.maximum(h, 0.0)


def _tc_layer2b(p_ref, cnt_ref, h_ref, wl_ref, wr_ref, b_ref, w3l_ref, w3r_ref,
                y_ref, z_ref):
    agg = jnp.concatenate([p_ref[0], p_ref[1]], axis=1) * _invd(cnt_ref)
    h2 = (jnp.dot(agg, wl_ref[...], preferred_element_type=jnp.float32)
          + jnp.dot(h_ref[...], wr_ref[...], preferred_element_type=jnp.float32)
          + b_ref[...])
    h2 = jnp.maximum(h2, 0.0)
    y_ref[...] = jnp.dot(h2, w3l_ref[...], preferred_element_type=jnp.float32)
    z_ref[...] = jnp.dot(h2, w3r_ref[...], preferred_element_type=jnp.float32)


def _tc_layer3(a_ref, cnt_ref, z_ref, b_ref, o_ref):
    h3 = ((a_ref[0, :, 0:1] + a_ref[1, :, 0:1]) * _invd(cnt_ref)
          + z_ref[:, 0:1] + b_ref[...])
    cols = lax.broadcasted_iota(jnp.int32, (NB, 16), 1)
    o_ref[...] = jnp.where(cols == 0, h3, 0.0)


def _tc_layer4(a_ref, cnt_ref, h3_ref, wl_ref, wr_ref, b_ref, o_ref):
    a4n = (a_ref[0, :, 0:1] + a_ref[1, :, 0:1]) * _invd(cnt_ref)
    logits = (a4n * wl_ref[...] + h3_ref[:, 0:1] * wr_ref[...] + b_ref[...])
    m = jnp.max(logits, axis=1, keepdims=True)
    sh = logits - m
    lse = jnp.log(jnp.sum(jnp.exp(sh), axis=1, keepdims=True))
    o_ref[...] = sh - lse


def _spec_a16():
    return pl.BlockSpec((2, NB, 16), lambda i: (0, i, 0))


def _spec_ph():
    return pl.BlockSpec((2, NB, HD), lambda i: (0, i, 0))


def _spec_rows(w):
    return pl.BlockSpec((NB, w), lambda i: (i, 0))


def _spec_full(shape):
    return pl.BlockSpec(shape, lambda i: tuple(0 for _ in shape))


def kernel(x, edge_index, Wl1, Wr1, b1, Wl2, Wr2, b2, Wl3, Wr3, b3,
           Wl4, Wr4, b4):
    x = x.astype(jnp.float32)
    src = edge_index[0].astype(jnp.int32)
    dst = edge_index[1].astype(jnp.int32)
    e = src.shape[0]
    quantum = NW * CH * NBUF
    e_pad = -(-e // quantum) * quantum
    if e_pad != e:
        src = jnp.concatenate([src, jnp.zeros((e_pad - e,), jnp.int32)])
        dst = jnp.concatenate(
            [dst, jnp.full((e_pad - e,), N_NODES, jnp.int32)])
    src = src.reshape(e_pad // CH, CH)
    dst = dst.reshape(e_pad // CH, CH)
    srcs_w = jnp.stack([src, src + NP])
    srcs_s = src[None]
    xp = jnp.pad(x, ((0, NP - x.shape[0]), (0, 0)))

    def halves(v):
        return v.reshape(NP, 2, HD).transpose(1, 0, 2).reshape(2 * NP, HD)

    zw = jnp.zeros((ROWS_PER_TILE, HD), jnp.float32)
    z16 = jnp.zeros((ROWS_PER_TILE, 16), jnp.float32)
    ones = jnp.ones((CH, 16), jnp.float32)

    agg_wide_cnt = _make_agg(e_pad, HD, True, True)
    agg_wide = _make_agg(e_pad, HD, True, False)
    agg_16 = _make_agg(e_pad, 16, False, False)

    p1, cnt = agg_wide_cnt(srcs_w, dst, halves(xp), zw, z16, ones)
    h1 = pl.pallas_call(
        _tc_layer12,
        grid=(GRID,),
        in_specs=[_spec_ph(), _spec_rows(16), _spec_rows(DIM),
                  _spec_full((DIM, DIM)), _spec_full((DIM, DIM)),
                  _spec_full((1, DIM))],
        out_specs=_spec_rows(DIM),
        out_shape=jax.ShapeDtypeStruct((NP, DIM), jnp.float32),
    )(p1.reshape(2, NP, HD), cnt, xp, Wl1, Wr1, b1.reshape(1, DIM))

    p2 = agg_wide(srcs_w, dst, halves(h1), zw)[0]
    w3l = jnp.pad(Wl3, ((0, 0), (0, 15)))
    w3r = jnp.pad(Wr3, ((0, 0), (0, 15)))
    y3w, z3w = pl.pallas_call(
        _tc_layer2b,
        grid=(GRID,),
        in_specs=[_spec_ph(), _spec_rows(16), _spec_rows(DIM),
                  _spec_full((DIM, DIM)), _spec_full((DIM, DIM)),
                  _spec_full((1, DIM)), _spec_full((DIM, 16)),
                  _spec_full((DIM, 16))],
        out_specs=[_spec_rows(16), _spec_rows(16)],
        out_shape=[jax.ShapeDtypeStruct((NP, 16), jnp.float32),
                   jax.ShapeDtypeStruct((NP, 16), jnp.float32)],
    )(p2.reshape(2, NP, HD), cnt, h1, Wl2, Wr2, b2.reshape(1, DIM), w3l, w3r)

    a3 = agg_16(srcs_s, dst, y3w, z16)[0]
    h3w = pl.pallas_call(
        _tc_layer3,
        grid=(GRID,),
        in_specs=[_spec_a16(), _spec_rows(16), _spec_rows(16),
                  _spec_full((1, 1))],
        out_specs=_spec_rows(16),
        out_shape=jax.ShapeDtypeStruct((NP, 16), jnp.float32),
    )(a3.reshape(2, NP, 16), cnt, z3w, b3.reshape(1, 1))

    a4 = agg_16(srcs_s, dst, h3w, z16)[0]
    out = pl.pallas_call(
        _tc_layer4,
        grid=(GRID,),
        in_specs=[_spec_a16(), _spec_rows(16), _spec_rows(16),
                  _spec_full((1, 16)), _spec_full((1, 16)),
                  _spec_full((1, 16))],
        out_specs=_spec_rows(16),
        out_shape=jax.ShapeDtypeStruct((NP, 16), jnp.float32),
    )(a4.reshape(2, NP, 16), cnt, h3w, Wl4, Wr4, b4.reshape(1, 16))

    n = x.shape[0]
    return (out[:n], h3w[:n, 0])
